# baseline (speedup 1.0000x reference)
import functools

import jax
import jax.numpy as jnp
from jax.experimental import pallas as pl
from jax.experimental.pallas import tpu as pltpu

_BLK = 2048


def _softplus(x):
    return jnp.maximum(x, 0.0) + jnp.log1p(jnp.exp(-jnp.abs(x)))


def _read_head_kernel(e2_ref, ws_ref, bs_ref, wsh_ref, bsh_ref, memt0_ref,
                      memt1_ref, out_ref, acc, zsum, mrun, embt_scr):
    i = pl.program_id(0)
    nb = pl.num_programs(0)

    @pl.when(i == 0)
    def _init():
        acc[...] = jnp.zeros_like(acc)
        zsum[...] = jnp.zeros_like(zsum)
        mrun[...] = jnp.full_like(mrun, -1e30)
        e2 = e2_ref[...]
        embt_scr[...] = jnp.transpose(
            e2.reshape(e2.shape[0] // 64, 64, e2.shape[1]), (2, 1, 0)
        ).reshape(e2.shape[0], e2.shape[1])

    embt = embt_scr[...]
    s_row = jax.lax.dot_general(ws_ref[...], embt, (((1,), (0,)), ((), ())),
                                preferred_element_type=jnp.float32) + bs_ref[0, 0]
    sh_row = jax.lax.dot_general(wsh_ref[...], embt, (((1,), (0,)), ((), ())),
                                 preferred_element_type=jnp.float32) + bsh_ref[0, 0]
    temp_row = _softplus(s_row) * (1.0 + _softplus(sh_row))
    ksq_row = jnp.sum(embt * embt, axis=0, keepdims=True)
    tscale_row = temp_row / (jnp.sqrt(ksq_row) + 1e-8)
    nb_b = embt.shape[1]
    eye = (jax.lax.broadcasted_iota(jnp.int32, (nb_b, nb_b), 0)
           == jax.lax.broadcasted_iota(jnp.int32, (nb_b, nb_b), 1))
    tscale = jnp.sum(jnp.where(eye, tscale_row, 0.0), axis=1, keepdims=True)

    m_old = mrun[...]
    logits_all = []
    m_new = m_old
    for mref in (memt0_ref, memt1_ref):
        memt = mref[...]
        raw = jax.lax.dot_general(embt, memt, (((0,), (0,)), ((), ())),
                                  preferred_element_type=jnp.float32)
        nsq = jnp.sum(memt * memt, axis=0, keepdims=True)
        inv = 1.0 / (jnp.sqrt(nsq) + 1e-8)
        logits = tscale * (raw * inv)
        logits_all.append(logits)
        m_new = jnp.maximum(m_new, jnp.max(logits, axis=1, keepdims=True))

    alpha = jnp.exp(m_old - m_new)
    z_part = jnp.zeros_like(zsum[...])
    a_part = jnp.zeros_like(acc[...])
    for mref, logits in zip((memt0_ref, memt1_ref), logits_all):
        p = jnp.exp(logits - m_new)
        z_part = z_part + jnp.sum(p, axis=1, keepdims=True)
        a_part = a_part + jax.lax.dot_general(
            p, mref[...], (((1,), (1,)), ((), ())),
            preferred_element_type=jnp.float32)
    zsum[...] = zsum[...] * alpha + z_part
    acc[...] = acc[...] * alpha + a_part
    mrun[...] = m_new

    @pl.when(i == nb - 1)
    def _fin():
        out_ref[...] = acc[...] / zsum[...]


@functools.partial(jax.jit, static_argnames=())
def kernel(embeddings, memory, W_strength, b_strength, W_sharpen, b_sharpen):
    B = embeddings.shape[0]
    N = memory.shape[0]
    D = memory.shape[1] * memory.shape[2] * memory.shape[3]
    e2 = embeddings.transpose(0, 2, 3, 1).reshape(D, B)
    memt = memory.transpose(1, 2, 3, 0).reshape(D, N)
    ws = W_strength.reshape(1, D)
    wsh = W_sharpen.reshape(1, D)
    bs = b_strength.reshape(1, 1)
    bsh = b_sharpen.reshape(1, 1)
    nb = N // (2 * _BLK)

    return pl.pallas_call(
        _read_head_kernel,
        grid=(nb,),
        in_specs=[
            pl.BlockSpec((D, B), lambda i: (0, 0)),
            pl.BlockSpec((1, D), lambda i: (0, 0)),
            pl.BlockSpec((1, 1), lambda i: (0, 0)),
            pl.BlockSpec((1, D), lambda i: (0, 0)),
            pl.BlockSpec((1, 1), lambda i: (0, 0)),
            pl.BlockSpec((D, _BLK), lambda i: (0, 2 * i)),
            pl.BlockSpec((D, _BLK), lambda i: (0, 2 * i + 1)),
        ],
        out_specs=pl.BlockSpec((B, D), lambda i: (0, 0)),
        out_shape=jax.ShapeDtypeStruct((B, D), jnp.float32),
        scratch_shapes=[
            pltpu.VMEM((B, D), jnp.float32),
            pltpu.VMEM((B, 1), jnp.float32),
            pltpu.VMEM((B, 1), jnp.float32),
            pltpu.VMEM((D, B), jnp.float32),
        ],
        compiler_params=pltpu.CompilerParams(
            dimension_semantics=("arbitrary",),
        ),
    )(e2, ws, bs, wsh, bsh, memt, memt)


# speedup vs baseline: 1.0934x; 1.0934x over previous
import functools

import jax
import jax.numpy as jnp
from jax.experimental import pallas as pl
from jax.experimental.pallas import tpu as pltpu

_BLK = 4096


def _softplus(x):
    return jnp.maximum(x, 0.0) + jnp.log1p(jnp.exp(-jnp.abs(x)))


def _read_head_kernel(e2_ref, ws_ref, bs_ref, wsh_ref, bsh_ref, memt_ref,
                      out_ref, acc, zsum, mrun, embt_scr):
    i = pl.program_id(0)
    nb = pl.num_programs(0)

    @pl.when(i == 0)
    def _init():
        acc[...] = jnp.zeros_like(acc)
        zsum[...] = jnp.zeros_like(zsum)
        mrun[...] = jnp.full_like(mrun, -1e30)
        e2 = e2_ref[...]
        embt_scr[...] = jnp.transpose(
            e2.reshape(e2.shape[0] // 64, 64, e2.shape[1]), (2, 1, 0)
        ).reshape(e2.shape[0], e2.shape[1])

    embt = embt_scr[...]
    s_row = jax.lax.dot_general(ws_ref[...], embt, (((1,), (0,)), ((), ())),
                                preferred_element_type=jnp.float32) + bs_ref[0, 0]
    sh_row = jax.lax.dot_general(wsh_ref[...], embt, (((1,), (0,)), ((), ())),
                                 preferred_element_type=jnp.float32) + bsh_ref[0, 0]
    temp_row = _softplus(s_row) * (1.0 + _softplus(sh_row))
    ksq_row = jnp.sum(embt * embt, axis=0, keepdims=True)
    tscale_row = temp_row / (jnp.sqrt(ksq_row) + 1e-8)
    nb_b = embt.shape[1]
    eye = (jax.lax.broadcasted_iota(jnp.int32, (nb_b, nb_b), 0)
           == jax.lax.broadcasted_iota(jnp.int32, (nb_b, nb_b), 1))
    tscale = jnp.sum(jnp.where(eye, tscale_row, 0.0), axis=1, keepdims=True)

    memt = memt_ref[...]
    raw = jax.lax.dot_general(embt, memt, (((0,), (0,)), ((), ())),
                              preferred_element_type=jnp.float32)
    nsq = jnp.sum(memt * memt, axis=0, keepdims=True)
    inv = 1.0 / (jnp.sqrt(nsq) + 1e-8)
    logits = tscale * (raw * inv)

    m_old = mrun[...]
    m_new = jnp.maximum(m_old, jnp.max(logits, axis=1, keepdims=True))
    alpha = jnp.exp(m_old - m_new)
    p = jnp.exp(logits - m_new)
    zsum[...] = zsum[...] * alpha + jnp.sum(p, axis=1, keepdims=True)
    acc[...] = acc[...] * alpha + jax.lax.dot_general(
        p, memt, (((1,), (1,)), ((), ())), preferred_element_type=jnp.float32)
    mrun[...] = m_new

    @pl.when(i == nb - 1)
    def _fin():
        out_ref[...] = acc[...] / zsum[...]


@functools.partial(jax.jit, static_argnames=())
def kernel(embeddings, memory, W_strength, b_strength, W_sharpen, b_sharpen):
    B = embeddings.shape[0]
    N = memory.shape[0]
    D = memory.shape[1] * memory.shape[2] * memory.shape[3]
    e2 = embeddings.transpose(0, 2, 3, 1).reshape(D, B)
    memt = memory.transpose(1, 2, 3, 0).reshape(D, N)
    ws = W_strength.reshape(1, D)
    wsh = W_sharpen.reshape(1, D)
    bs = b_strength.reshape(1, 1)
    bsh = b_sharpen.reshape(1, 1)
    nb = N // _BLK

    return pl.pallas_call(
        _read_head_kernel,
        grid=(nb,),
        in_specs=[
            pl.BlockSpec((D, B), lambda i: (0, 0)),
            pl.BlockSpec((1, D), lambda i: (0, 0)),
            pl.BlockSpec((1, 1), lambda i: (0, 0)),
            pl.BlockSpec((1, D), lambda i: (0, 0)),
            pl.BlockSpec((1, 1), lambda i: (0, 0)),
            pl.BlockSpec((D, _BLK), lambda i: (0, i)),
        ],
        out_specs=pl.BlockSpec((B, D), lambda i: (0, 0)),
        out_shape=jax.ShapeDtypeStruct((B, D), jnp.float32),
        scratch_shapes=[
            pltpu.VMEM((B, D), jnp.float32),
            pltpu.VMEM((B, 1), jnp.float32),
            pltpu.VMEM((B, 1), jnp.float32),
            pltpu.VMEM((D, B), jnp.float32),
        ],
        compiler_params=pltpu.CompilerParams(
            dimension_semantics=("arbitrary",),
        ),
    )(e2, ws, bs, wsh, bsh, memt)
